# baseline (speedup 1.0000x reference)
import functools

import jax
import jax.numpy as jnp
from jax.experimental import pallas as pl
from jax.experimental.pallas import tpu as pltpu

_HIGHEST = jax.lax.Precision.HIGHEST


def _dot(a, b):
    return jax.lax.dot_general(a, b, (((1,), (0,)), ((), ())),
                               precision=_HIGHEST,
                               preferred_element_type=jnp.float32)


def _dotd(a, b):
    return jax.lax.dot_general(a, b, (((1,), (0,)), ((), ())),
                               precision=jax.lax.Precision.DEFAULT,
                               preferred_element_type=jnp.float32)


def _knn_edgeconv_kernel(pos_blk_ref, pos_all_ref, posT_ref,
                         W1_ref, b1_ref, W2_ref, b2_ref, W3_ref, b3_ref,
                         pw_ref, pb_ref, out_ref, e_scr, idx_scr, *, R, N, K):
    q = pos_blk_ref[0]
    p = pos_all_ref[0]

    d = jnp.zeros((R, N), dtype=jnp.float32)
    for c in range(3):
        qc = q[:, c:c + 1]
        pc = posT_ref[0, c, :].reshape(1, N)
        diff = qc - pc
        d = d + diff * diff

    col = jax.lax.broadcasted_iota(jnp.int32, (R, N), 1)

    W1 = W1_ref[...]
    b1 = b1_ref[...]
    W2 = W2_ref[...]
    b2 = b2_ref[...]
    W3 = W3_ref[...]
    b3 = b3_ref[...]

    for k in range(K + 1):
        m = jnp.min(d, axis=1, keepdims=True)
        cand = jnp.where(d == m, col, N)
        idx = jnp.min(cand, axis=1, keepdims=True)
        d = jnp.where(col == idx, jnp.inf, d)
        if k == 0:
            continue
        idx_scr[(k - 1) * R:k * R, :] = idx

    def _gather(k, carry):
        idx_k = idx_scr[pl.ds(k * R, R), :]
        sel = (col == idx_k).astype(jnp.float32)
        nb = _dot(sel, p)
        e_scr[pl.ds(k * R, R), 0:3] = q
        e_scr[pl.ds(k * R, R), 3:6] = nb
        e_scr[pl.ds(k * R, R), 6:9] = nb - q
        return carry

    jax.lax.fori_loop(0, K, _gather, 0)

    e = e_scr[...]
    qt = e[:, 0:3]
    y1r = jax.nn.relu(_dotd(e, W1) + b1)
    y1 = jnp.concatenate([y1r, qt], axis=1)
    y2r = jax.nn.relu(_dotd(y1, W2) + b2)
    y2 = jnp.concatenate([y2r, y1], axis=1)
    l3 = _dotd(y2, W3) + b3
    y3 = jnp.concatenate([l3, y2], axis=1)
    fmax = jnp.max(y3.reshape(K, R, 99), axis=0)

    pw = pw_ref[...]
    norm = jnp.sqrt(jnp.sum(pw * pw))
    score = (_dotd(fmax, pw) + pb_ref[0, 0]) / norm
    out_ref[0] = jnp.concatenate([fmax, score], axis=1)


def _gpool_adjust_kernel(feat_ref, pos_ref, srow_ref, scol_ref,
                         m1w_ref, m1b_ref, m2w_ref, m2b_ref,
                         m3w_ref, m3b_ref, out_ref, *, N, TOPK, CH):
    s_row = srow_ref[0]
    irow = jax.lax.broadcasted_iota(jnp.int32, (1, N), 1)

    rank = jnp.zeros((1, N), dtype=jnp.int32)
    for c in range(N // CH):
        s_col = scol_ref[0, c * CH:(c + 1) * CH, :]
        jcol = (jax.lax.broadcasted_iota(jnp.int32, (CH, 1), 0)
                + c * CH)
        gt = s_col > s_row
        eq = (s_col == s_row) & (jcol < irow)
        cnt = (gt | eq).astype(jnp.int32)
        rank = rank + jnp.sum(cnt, axis=0, keepdims=True)

    pos = pos_ref[0]
    feat = feat_ref[0, :, :99]
    s_colv = scol_ref[0]

    m1w = m1w_ref[...]
    m1b = m1b_ref[...]
    m2w = m2w_ref[...]
    m2b = m2b_ref[...]
    m3w = m3w_ref[...]
    m3b = m3b_ref[...]

    for r0 in range(0, TOPK, CH):
        rr = (jax.lax.broadcasted_iota(jnp.int32, (CH, 1), 0) + r0)
        P = (rank == rr).astype(jnp.float32)
        pos_sel = _dot(P, pos)
        feat_sel = _dot(P, feat)
        s_sel = _dot(P, s_colv)
        gate = jax.nn.sigmoid(s_sel)
        x_ds = feat_sel * gate
        h = jax.nn.relu(_dotd(x_ds, m1w) + m1b)
        h = jax.nn.relu(_dotd(h, m2w) + m2b)
        adj = _dotd(h, m3w) + m3b
        out_ref[0, r0:r0 + CH, :] = pos_sel + adj


def kernel(pos, W1, b1, W2, b2, W3, b3, pw, pb, m1w, m1b, m2w, m2b, m3w, m3b):
    B, N, _ = pos.shape
    K = 16
    R = 256
    TOPK = N // 2
    CH = 256

    posT = jnp.transpose(pos, (0, 2, 1))

    feat100 = pl.pallas_call(
        functools.partial(_knn_edgeconv_kernel, R=R, N=N, K=K),
        grid=(B, N // R),
        in_specs=[
            pl.BlockSpec((1, R, 3), lambda b, n: (b, n, 0)),
            pl.BlockSpec((1, N, 3), lambda b, n: (b, 0, 0)),
            pl.BlockSpec((1, 3, N), lambda b, n: (b, 0, 0)),
            pl.BlockSpec((9, 32), lambda b, n: (0, 0)),
            pl.BlockSpec((1, 32), lambda b, n: (0, 0)),
            pl.BlockSpec((35, 32), lambda b, n: (0, 0)),
            pl.BlockSpec((1, 32), lambda b, n: (0, 0)),
            pl.BlockSpec((67, 32), lambda b, n: (0, 0)),
            pl.BlockSpec((1, 32), lambda b, n: (0, 0)),
            pl.BlockSpec((99, 1), lambda b, n: (0, 0)),
            pl.BlockSpec((1, 1), lambda b, n: (0, 0)),
        ],
        out_specs=pl.BlockSpec((1, R, 100), lambda b, n: (b, n, 0)),
        out_shape=jax.ShapeDtypeStruct((B, N, 100), jnp.float32),
        scratch_shapes=[pltpu.VMEM((K * R, 9), jnp.float32),
                        pltpu.VMEM((K * R, 1), jnp.int32)],
    )(pos, pos, posT,
      W1, b1.reshape(1, 32), W2, b2.reshape(1, 32), W3, b3.reshape(1, 32),
      pw, pb.reshape(1, 1))

    score = feat100[:, :, 99]
    s_row = score.reshape(B, 1, N)
    s_col = score.reshape(B, N, 1)

    out = pl.pallas_call(
        functools.partial(_gpool_adjust_kernel, N=N, TOPK=TOPK, CH=CH),
        grid=(B,),
        in_specs=[
            pl.BlockSpec((1, N, 100), lambda b: (b, 0, 0)),
            pl.BlockSpec((1, N, 3), lambda b: (b, 0, 0)),
            pl.BlockSpec((1, 1, N), lambda b: (b, 0, 0)),
            pl.BlockSpec((1, N, 1), lambda b: (b, 0, 0)),
            pl.BlockSpec((99, 49), lambda b: (0, 0)),
            pl.BlockSpec((1, 49), lambda b: (0, 0)),
            pl.BlockSpec((49, 24), lambda b: (0, 0)),
            pl.BlockSpec((1, 24), lambda b: (0, 0)),
            pl.BlockSpec((24, 3), lambda b: (0, 0)),
            pl.BlockSpec((1, 3), lambda b: (0, 0)),
        ],
        out_specs=pl.BlockSpec((1, TOPK, 3), lambda b: (b, 0, 0)),
        out_shape=jax.ShapeDtypeStruct((B, TOPK, 3), jnp.float32),
    )(feat100, pos, s_row, s_col,
      m1w, m1b.reshape(1, 49), m2w, m2b.reshape(1, 24), m3w, m3b.reshape(1, 3))

    return out


# speedup vs baseline: 1.6318x; 1.6318x over previous
import functools

import jax
import jax.numpy as jnp
from jax.experimental import pallas as pl
from jax.experimental.pallas import tpu as pltpu

_HIGHEST = jax.lax.Precision.HIGHEST


def _dot(a, b):
    return jax.lax.dot_general(a, b, (((1,), (0,)), ((), ())),
                               precision=_HIGHEST,
                               preferred_element_type=jnp.float32)


def _dotd(a, b):
    return jax.lax.dot_general(a, b, (((1,), (0,)), ((), ())),
                               precision=jax.lax.Precision.DEFAULT,
                               preferred_element_type=jnp.float32)


def _knn_edgeconv_kernel(pos_blk_ref, pos_all_ref, posT_ref,
                         W1_ref, b1_ref, W2_ref, b2_ref, W3_ref, b3_ref,
                         pw_ref, pb_ref, out_ref, e_scr, *, R, N, K):
    q = pos_blk_ref[0]
    p = pos_all_ref[0]

    d = jnp.zeros((R, N), dtype=jnp.float32)
    for c in range(3):
        qc = q[:, c:c + 1]
        pc = posT_ref[0, c, :].reshape(1, N)
        diff = qc - pc
        d = d + diff * diff

    col = jax.lax.broadcasted_iota(jnp.int32, (R, N), 1)

    W1 = W1_ref[...]
    b1 = b1_ref[...]
    W2 = W2_ref[...]
    b2 = b2_ref[...]
    W3 = W3_ref[...]
    b3 = b3_ref[...]

    pcs = [posT_ref[0, c, :].reshape(1, N) for c in range(3)]
    for k in range(K + 1):
        m = jnp.min(d, axis=1, keepdims=True)
        cand = jnp.where(d == m, col, N)
        idx = jnp.min(cand, axis=1, keepdims=True)
        sel = col == idx
        d = jnp.where(sel, jnp.inf, d)
        if k == 0:
            continue
        nb = jnp.concatenate(
            [jnp.min(jnp.where(sel, pc, jnp.inf), axis=1, keepdims=True)
             for pc in pcs], axis=1)
        e_scr[(k - 1) * R:k * R, :] = jnp.concatenate([q, nb, nb - q], axis=1)

    e = e_scr[...]
    qt = e[:, 0:3]
    y1r = jax.nn.relu(_dotd(e, W1) + b1)
    y1 = jnp.concatenate([y1r, qt], axis=1)
    y2r = jax.nn.relu(_dotd(y1, W2) + b2)
    y2 = jnp.concatenate([y2r, y1], axis=1)
    l3 = _dotd(y2, W3) + b3
    y3 = jnp.concatenate([l3, y2], axis=1)
    fmax = jnp.max(y3.reshape(K, R, 99), axis=0)

    pw = pw_ref[...]
    norm = jnp.sqrt(jnp.sum(pw * pw))
    score = (_dotd(fmax, pw) + pb_ref[0, 0]) / norm
    out_ref[0] = jnp.concatenate([fmax, score], axis=1)


def _gpool_adjust_kernel(feat_ref, pos_ref, srow_ref, scol_ref,
                         m1w_ref, m1b_ref, m2w_ref, m2b_ref,
                         m3w_ref, m3b_ref, out_ref, *, N, TOPK, CH):
    s_row = srow_ref[0]
    irow = jax.lax.broadcasted_iota(jnp.int32, (1, N), 1)

    rank = jnp.zeros((1, N), dtype=jnp.int32)
    for c in range(N // CH):
        s_col = scol_ref[0, c * CH:(c + 1) * CH, :]
        jcol = (jax.lax.broadcasted_iota(jnp.int32, (CH, 1), 0)
                + c * CH)
        gt = s_col > s_row
        eq = (s_col == s_row) & (jcol < irow)
        cnt = (gt | eq).astype(jnp.int32)
        rank = rank + jnp.sum(cnt, axis=0, keepdims=True)

    pos = pos_ref[0]
    feat = feat_ref[0, :, :99]
    s_colv = scol_ref[0]

    m1w = m1w_ref[...]
    m1b = m1b_ref[...]
    m2w = m2w_ref[...]
    m2b = m2b_ref[...]
    m3w = m3w_ref[...]
    m3b = m3b_ref[...]

    for r0 in range(0, TOPK, CH):
        rr = (jax.lax.broadcasted_iota(jnp.int32, (CH, 1), 0) + r0)
        P = (rank == rr).astype(jnp.float32)
        pos_sel = _dot(P, pos)
        feat_sel = _dot(P, feat)
        s_sel = _dot(P, s_colv)
        gate = jax.nn.sigmoid(s_sel)
        x_ds = feat_sel * gate
        h = jax.nn.relu(_dotd(x_ds, m1w) + m1b)
        h = jax.nn.relu(_dotd(h, m2w) + m2b)
        adj = _dotd(h, m3w) + m3b
        out_ref[0, r0:r0 + CH, :] = pos_sel + adj


def kernel(pos, W1, b1, W2, b2, W3, b3, pw, pb, m1w, m1b, m2w, m2b, m3w, m3b):
    B, N, _ = pos.shape
    K = 16
    R = 512
    TOPK = N // 2
    CH = 256

    posT = jnp.transpose(pos, (0, 2, 1))

    feat100 = pl.pallas_call(
        functools.partial(_knn_edgeconv_kernel, R=R, N=N, K=K),
        grid=(B, N // R),
        in_specs=[
            pl.BlockSpec((1, R, 3), lambda b, n: (b, n, 0)),
            pl.BlockSpec((1, N, 3), lambda b, n: (b, 0, 0)),
            pl.BlockSpec((1, 3, N), lambda b, n: (b, 0, 0)),
            pl.BlockSpec((9, 32), lambda b, n: (0, 0)),
            pl.BlockSpec((1, 32), lambda b, n: (0, 0)),
            pl.BlockSpec((35, 32), lambda b, n: (0, 0)),
            pl.BlockSpec((1, 32), lambda b, n: (0, 0)),
            pl.BlockSpec((67, 32), lambda b, n: (0, 0)),
            pl.BlockSpec((1, 32), lambda b, n: (0, 0)),
            pl.BlockSpec((99, 1), lambda b, n: (0, 0)),
            pl.BlockSpec((1, 1), lambda b, n: (0, 0)),
        ],
        out_specs=pl.BlockSpec((1, R, 100), lambda b, n: (b, n, 0)),
        out_shape=jax.ShapeDtypeStruct((B, N, 100), jnp.float32),
        scratch_shapes=[pltpu.VMEM((K * R, 9), jnp.float32)],
    )(pos, pos, posT,
      W1, b1.reshape(1, 32), W2, b2.reshape(1, 32), W3, b3.reshape(1, 32),
      pw, pb.reshape(1, 1))

    score = feat100[:, :, 99]
    s_row = score.reshape(B, 1, N)
    s_col = score.reshape(B, N, 1)

    out = pl.pallas_call(
        functools.partial(_gpool_adjust_kernel, N=N, TOPK=TOPK, CH=CH),
        grid=(B,),
        in_specs=[
            pl.BlockSpec((1, N, 100), lambda b: (b, 0, 0)),
            pl.BlockSpec((1, N, 3), lambda b: (b, 0, 0)),
            pl.BlockSpec((1, 1, N), lambda b: (b, 0, 0)),
            pl.BlockSpec((1, N, 1), lambda b: (b, 0, 0)),
            pl.BlockSpec((99, 49), lambda b: (0, 0)),
            pl.BlockSpec((1, 49), lambda b: (0, 0)),
            pl.BlockSpec((49, 24), lambda b: (0, 0)),
            pl.BlockSpec((1, 24), lambda b: (0, 0)),
            pl.BlockSpec((24, 3), lambda b: (0, 0)),
            pl.BlockSpec((1, 3), lambda b: (0, 0)),
        ],
        out_specs=pl.BlockSpec((1, TOPK, 3), lambda b: (b, 0, 0)),
        out_shape=jax.ShapeDtypeStruct((B, TOPK, 3), jnp.float32),
    )(feat100, pos, s_row, s_col,
      m1w, m1b.reshape(1, 49), m2w, m2b.reshape(1, 24), m3w, m3b.reshape(1, 3))

    return out


# speedup vs baseline: 1.8477x; 1.1323x over previous
import functools

import jax
import jax.numpy as jnp
from jax import lax
from jax.experimental import pallas as pl
from jax.experimental.pallas import tpu as pltpu
from jax.experimental.pallas import tpu_sc as plsc

_HIGHEST = jax.lax.Precision.HIGHEST


def _dot(a, b):
    return jax.lax.dot_general(a, b, (((1,), (0,)), ((), ())),
                               precision=_HIGHEST,
                               preferred_element_type=jnp.float32)


def _dotd(a, b):
    return jax.lax.dot_general(a, b, (((1,), (0,)), ((), ())),
                               precision=jax.lax.Precision.DEFAULT,
                               preferred_element_type=jnp.float32)


def _knn_edgeconv_kernel(pos_blk_ref, pos_all_ref, posT_ref,
                         W1_ref, b1_ref, W2_ref, b2_ref, W3_ref, b3_ref,
                         pw_ref, pb_ref, out_ref, e_scr, *, R, N, K):
    q = pos_blk_ref[0]
    p = pos_all_ref[0]

    d = jnp.zeros((R, N), dtype=jnp.float32)
    for c in range(3):
        qc = q[:, c:c + 1]
        pc = posT_ref[0, c, :].reshape(1, N)
        diff = qc - pc
        d = d + diff * diff

    col = jax.lax.broadcasted_iota(jnp.int32, (R, N), 1)

    W1 = W1_ref[...]
    b1 = b1_ref[...]
    W2 = W2_ref[...]
    b2 = b2_ref[...]
    W3 = W3_ref[...]
    b3 = b3_ref[...]

    pcs = [posT_ref[0, c, :].reshape(1, N) for c in range(3)]
    for k in range(K + 1):
        m = jnp.min(d, axis=1, keepdims=True)
        cand = jnp.where(d == m, col, N)
        idx = jnp.min(cand, axis=1, keepdims=True)
        sel = col == idx
        d = jnp.where(sel, jnp.inf, d)
        if k == 0:
            continue
        nb = jnp.concatenate(
            [jnp.min(jnp.where(sel, pc, jnp.inf), axis=1, keepdims=True)
             for pc in pcs], axis=1)
        e_scr[(k - 1) * R:k * R, :] = jnp.concatenate([q, nb, nb - q], axis=1)

    e = e_scr[...]
    qt = e[:, 0:3]
    y1r = jax.nn.relu(_dotd(e, W1) + b1)
    y1 = jnp.concatenate([y1r, qt], axis=1)
    y2r = jax.nn.relu(_dotd(y1, W2) + b2)
    y2 = jnp.concatenate([y2r, y1], axis=1)
    l3 = _dotd(y2, W3) + b3
    y3 = jnp.concatenate([l3, y2], axis=1)
    fmax = jnp.max(y3.reshape(K, R, 99), axis=0)

    pw = pw_ref[...]
    norm = jnp.sqrt(jnp.sum(pw * pw))
    score = (_dotd(fmax, pw) + pb_ref[0, 0]) / norm
    pad = jnp.zeros((R, 25), dtype=jnp.float32)
    out_ref[0] = jnp.concatenate([fmax, score, q, pad], axis=1)


def _rank_kernel(srow_ref, scol_ref, rank_ref, *, N, CH):
    s_row = srow_ref[0]
    irow = jax.lax.broadcasted_iota(jnp.int32, (1, N), 1)
    rank = jnp.zeros((1, N), dtype=jnp.int32)
    for c in range(N // CH):
        s_col = scol_ref[0, c * CH:(c + 1) * CH, :]
        jcol = (jax.lax.broadcasted_iota(jnp.int32, (CH, 1), 0)
                + c * CH)
        gt = s_col > s_row
        eq = (s_col == s_row) & (jcol < irow)
        cnt = (gt | eq).astype(jnp.int32)
        rank = rank + jnp.sum(cnt, axis=0, keepdims=True)
    rank_ref[0] = rank


def _make_sc_scatter(B, N):
    mesh = plsc.VectorSubcoreMesh(core_axis_name="c", subcore_axis_name="s")
    per_w = (B * N) // 32
    n_ch = per_w // 128

    @functools.partial(
        pl.kernel, mesh=mesh,
        out_type=jax.ShapeDtypeStruct((B * N,), jnp.int32),
        scratch_types=[
            pltpu.VMEM((128,), jnp.int32),
            pltpu.VMEM((128,), jnp.int32),
            pltpu.SemaphoreType.DMA,
        ],
    )
    def sc_scatter(rank_hbm, inv_hbm, idx_v, val_v, sem):
        wid = lax.axis_index("s") * 2 + lax.axis_index("c")
        base = wid * per_w
        for ch in range(n_ch):
            off = base + ch * 128
            pltpu.sync_copy(rank_hbm.at[pl.ds(off, 128)], idx_v)
            b = (off // N) * N
            for j in range(8):
                lane = jax.lax.iota(jnp.int32, 16)
                val_v[pl.ds(j * 16, 16)] = lane + (off + j * 16)
                idx_v[pl.ds(j * 16, 16)] = idx_v[pl.ds(j * 16, 16)] + b
            pltpu.async_copy(val_v, inv_hbm.at[idx_v], sem).wait()

    return sc_scatter


def _make_sc_gather(B, N, TOPK, D):
    mesh = plsc.VectorSubcoreMesh(core_axis_name="c", subcore_axis_name="s")
    per_w = (B * TOPK) // 32
    n_ch = per_w // 128

    @functools.partial(
        pl.kernel, mesh=mesh,
        out_type=jax.ShapeDtypeStruct((B * TOPK, D), jnp.float32),
        scratch_types=[
            pltpu.VMEM((128,), jnp.int32),
            pltpu.VMEM((128, D), jnp.float32),
            pltpu.SemaphoreType.DMA,
        ],
    )
    def sc_gather(inv_hbm, table_hbm, out_hbm, idx_v, rows_v, sem):
        wid = lax.axis_index("s") * 2 + lax.axis_index("c")
        for ch in range(n_ch):
            o = wid * per_w + ch * 128
            b = o // TOPK
            r = o - b * TOPK
            pltpu.sync_copy(inv_hbm.at[pl.ds(b * N + r, 128)], idx_v)
            pltpu.async_copy(table_hbm.at[idx_v], rows_v, sem).wait()
            pltpu.sync_copy(rows_v, out_hbm.at[pl.ds(o, 128)])

    return sc_gather


def _gate_adjust_kernel(sel_ref, m1w_ref, m1b_ref, m2w_ref, m2b_ref,
                        m3w_ref, m3b_ref, out_ref, *, TOPK):
    x = sel_ref[0]
    feat = x[:, 0:99]
    score = x[:, 99:100]
    pos_sel = x[:, 100:103]
    gate = jax.nn.sigmoid(score)
    x_ds = feat * gate
    h = jax.nn.relu(_dotd(x_ds, m1w_ref[...]) + m1b_ref[...])
    h = jax.nn.relu(_dotd(h, m2w_ref[...]) + m2b_ref[...])
    adj = _dotd(h, m3w_ref[...]) + m3b_ref[...]
    out_ref[0] = pos_sel + adj


def kernel(pos, W1, b1, W2, b2, W3, b3, pw, pb, m1w, m1b, m2w, m2b, m3w, m3b):
    B, N, _ = pos.shape
    K = 16
    R = 512
    TOPK = N // 2
    CH = 256

    posT = jnp.transpose(pos, (0, 2, 1))

    table = pl.pallas_call(
        functools.partial(_knn_edgeconv_kernel, R=R, N=N, K=K),
        grid=(B, N // R),
        in_specs=[
            pl.BlockSpec((1, R, 3), lambda b, n: (b, n, 0)),
            pl.BlockSpec((1, N, 3), lambda b, n: (b, 0, 0)),
            pl.BlockSpec((1, 3, N), lambda b, n: (b, 0, 0)),
            pl.BlockSpec((9, 32), lambda b, n: (0, 0)),
            pl.BlockSpec((1, 32), lambda b, n: (0, 0)),
            pl.BlockSpec((35, 32), lambda b, n: (0, 0)),
            pl.BlockSpec((1, 32), lambda b, n: (0, 0)),
            pl.BlockSpec((67, 32), lambda b, n: (0, 0)),
            pl.BlockSpec((1, 32), lambda b, n: (0, 0)),
            pl.BlockSpec((99, 1), lambda b, n: (0, 0)),
            pl.BlockSpec((1, 1), lambda b, n: (0, 0)),
        ],
        out_specs=pl.BlockSpec((1, R, 128), lambda b, n: (b, n, 0)),
        out_shape=jax.ShapeDtypeStruct((B, N, 128), jnp.float32),
        scratch_shapes=[pltpu.VMEM((K * R, 9), jnp.float32)],
    )(pos, pos, posT,
      W1, b1.reshape(1, 32), W2, b2.reshape(1, 32), W3, b3.reshape(1, 32),
      pw, pb.reshape(1, 1))

    score = table[:, :, 99]
    s_row = score.reshape(B, 1, N)
    s_col = score.reshape(B, N, 1)

    rank = pl.pallas_call(
        functools.partial(_rank_kernel, N=N, CH=CH),
        grid=(B,),
        in_specs=[
            pl.BlockSpec((1, 1, N), lambda b: (b, 0, 0)),
            pl.BlockSpec((1, N, 1), lambda b: (b, 0, 0)),
        ],
        out_specs=pl.BlockSpec((1, 1, N), lambda b: (b, 0, 0)),
        out_shape=jax.ShapeDtypeStruct((B, 1, N), jnp.int32),
    )(s_row, s_col)

    inv = _make_sc_scatter(B, N)(rank.reshape(B * N))
    sel = _make_sc_gather(B, N, TOPK, 128)(inv, table.reshape(B * N, 128))
    sel = sel.reshape(B, TOPK, 128)

    out = pl.pallas_call(
        functools.partial(_gate_adjust_kernel, TOPK=TOPK),
        grid=(B,),
        in_specs=[
            pl.BlockSpec((1, TOPK, 128), lambda b: (b, 0, 0)),
            pl.BlockSpec((99, 49), lambda b: (0, 0)),
            pl.BlockSpec((1, 49), lambda b: (0, 0)),
            pl.BlockSpec((49, 24), lambda b: (0, 0)),
            pl.BlockSpec((1, 24), lambda b: (0, 0)),
            pl.BlockSpec((24, 3), lambda b: (0, 0)),
            pl.BlockSpec((1, 3), lambda b: (0, 0)),
        ],
        out_specs=pl.BlockSpec((1, TOPK, 3), lambda b: (b, 0, 0)),
        out_shape=jax.ShapeDtypeStruct((B, TOPK, 3), jnp.float32),
    )(sel, m1w, m1b.reshape(1, 49), m2w, m2b.reshape(1, 24),
      m3w, m3b.reshape(1, 3))

    return out


# speedup vs baseline: 2.8531x; 1.5442x over previous
import functools

import jax
import jax.numpy as jnp
from jax import lax
from jax.experimental import pallas as pl
from jax.experimental.pallas import tpu as pltpu
from jax.experimental.pallas import tpu_sc as plsc

_HIGHEST = jax.lax.Precision.HIGHEST


def _dot(a, b):
    return jax.lax.dot_general(a, b, (((1,), (0,)), ((), ())),
                               precision=_HIGHEST,
                               preferred_element_type=jnp.float32)


def _dotd(a, b):
    return jax.lax.dot_general(a, b, (((1,), (0,)), ((), ())),
                               precision=jax.lax.Precision.DEFAULT,
                               preferred_element_type=jnp.float32)


def _knn_extract_kernel(pos_blk_ref, posT_ref, idx_ref, *, R, N, K):
    q = pos_blk_ref[0]

    d = jnp.zeros((R, N), dtype=jnp.float32)
    for c in range(3):
        qc = q[:, c:c + 1]
        pc = posT_ref[0, c, :].reshape(1, N)
        diff = qc - pc
        d = d + diff * diff

    col = jax.lax.broadcasted_iota(jnp.int32, (R, N), 1)
    boff = pl.program_id(0) * N

    for k in range(K + 1):
        m = jnp.min(d, axis=1, keepdims=True)
        cand = jnp.where(d == m, col, N)
        idx = jnp.min(cand, axis=1, keepdims=True)
        d = jnp.where(col == idx, jnp.inf, d)
        if k == 0:
            continue
        idx_ref[0, :, k - 1:k] = idx + boff


def _edgeconv_kernel(pos_blk_ref, nb_ref,
                     W1_ref, b1_ref, W2_ref, b2_ref, W3_ref, b3_ref,
                     pw_ref, pb_ref, out_ref, *, R, N, K):
    q = pos_blk_ref[0]
    W1 = W1_ref[...]
    b1 = b1_ref[...]
    W2 = W2_ref[...]
    b2 = b2_ref[...]
    W3 = W3_ref[...]
    b3 = b3_ref[...]

    nb = nb_ref[:, 0, :, 0:3].reshape(K * R, 3)
    qt = jnp.broadcast_to(q[None], (K, R, 3)).reshape(K * R, 3)
    e = jnp.concatenate([qt, nb, nb - qt], axis=1)
    y1r = jax.nn.relu(_dotd(e, W1) + b1)
    y1 = jnp.concatenate([y1r, qt], axis=1)
    y2r = jax.nn.relu(_dotd(y1, W2) + b2)
    y2 = jnp.concatenate([y2r, y1], axis=1)
    l3 = _dotd(y2, W3) + b3
    y3 = jnp.concatenate([l3, y2], axis=1)
    fmax = jnp.max(y3.reshape(K, R, 99), axis=0)

    pw = pw_ref[...]
    norm = jnp.sqrt(jnp.sum(pw * pw))
    score = (_dotd(fmax, pw) + pb_ref[0, 0]) / norm
    pad = jnp.zeros((R, 25), dtype=jnp.float32)
    out_ref[0] = jnp.concatenate([fmax, score, q, pad], axis=1)


def _rank_kernel(srow_ref, scol_ref, rank_ref, *, N, CH):
    s_row = srow_ref[0]
    irow = jax.lax.broadcasted_iota(jnp.int32, (1, N), 1)
    rank = jnp.zeros((1, N), dtype=jnp.int32)
    for c in range(N // CH):
        s_col = scol_ref[0, c * CH:(c + 1) * CH, :]
        jcol = (jax.lax.broadcasted_iota(jnp.int32, (CH, 1), 0)
                + c * CH)
        gt = s_col > s_row
        eq = (s_col == s_row) & (jcol < irow)
        cnt = (gt | eq).astype(jnp.int32)
        rank = rank + jnp.sum(cnt, axis=0, keepdims=True)
    rank_ref[0] = rank


def _make_sc_scatter(B, N):
    mesh = plsc.VectorSubcoreMesh(core_axis_name="c", subcore_axis_name="s")
    per_w = (B * N) // 32
    n_ch = per_w // 128

    @functools.partial(
        pl.kernel, mesh=mesh,
        out_type=jax.ShapeDtypeStruct((B * N,), jnp.int32),
        scratch_types=[
            pltpu.VMEM((128,), jnp.int32),
            pltpu.VMEM((128,), jnp.int32),
            pltpu.SemaphoreType.DMA,
        ],
    )
    def sc_scatter(rank_hbm, inv_hbm, idx_v, val_v, sem):
        wid = lax.axis_index("s") * 2 + lax.axis_index("c")
        base = wid * per_w
        for ch in range(n_ch):
            off = base + ch * 128
            pltpu.sync_copy(rank_hbm.at[pl.ds(off, 128)], idx_v)
            b = (off // N) * N
            for j in range(8):
                lane = jax.lax.iota(jnp.int32, 16)
                val_v[pl.ds(j * 16, 16)] = lane + (off + j * 16)
                idx_v[pl.ds(j * 16, 16)] = idx_v[pl.ds(j * 16, 16)] + b
            pltpu.async_copy(val_v, inv_hbm.at[idx_v], sem).wait()

    return sc_scatter


def _make_sc_gather_flat(M, D):
    mesh = plsc.VectorSubcoreMesh(core_axis_name="c", subcore_axis_name="s")
    per_w = M // 32

    @functools.partial(
        pl.kernel, mesh=mesh,
        out_type=jax.ShapeDtypeStruct((M, D), jnp.float32),
        scratch_types=[
            pltpu.VMEM((128,), jnp.int32),
            pltpu.VMEM((128, D), jnp.float32),
            pltpu.SemaphoreType.DMA,
        ],
    )
    def sc_gather_flat(idx_hbm, table_hbm, out_hbm, idx_v, rows_v, sem):
        wid = lax.axis_index("s") * 2 + lax.axis_index("c")
        base = wid * per_w

        def body(ch, carry):
            o = base + ch * 128
            pltpu.sync_copy(idx_hbm.at[pl.ds(o, 128)], idx_v)
            pltpu.async_copy(table_hbm.at[idx_v], rows_v, sem).wait()
            pltpu.sync_copy(rows_v, out_hbm.at[pl.ds(o, 128)])
            return carry

        lax.fori_loop(0, per_w // 128, body, 0)

    return sc_gather_flat


def _make_sc_gather(B, N, TOPK, D):
    mesh = plsc.VectorSubcoreMesh(core_axis_name="c", subcore_axis_name="s")
    per_w = (B * TOPK) // 32
    n_ch = per_w // 128

    @functools.partial(
        pl.kernel, mesh=mesh,
        out_type=jax.ShapeDtypeStruct((B * TOPK, D), jnp.float32),
        scratch_types=[
            pltpu.VMEM((128,), jnp.int32),
            pltpu.VMEM((128, D), jnp.float32),
            pltpu.SemaphoreType.DMA,
        ],
    )
    def sc_gather(inv_hbm, table_hbm, out_hbm, idx_v, rows_v, sem):
        wid = lax.axis_index("s") * 2 + lax.axis_index("c")
        for ch in range(n_ch):
            o = wid * per_w + ch * 128
            b = o // TOPK
            r = o - b * TOPK
            pltpu.sync_copy(inv_hbm.at[pl.ds(b * N + r, 128)], idx_v)
            pltpu.async_copy(table_hbm.at[idx_v], rows_v, sem).wait()
            pltpu.sync_copy(rows_v, out_hbm.at[pl.ds(o, 128)])

    return sc_gather


def _gate_adjust_kernel(sel_ref, m1w_ref, m1b_ref, m2w_ref, m2b_ref,
                        m3w_ref, m3b_ref, out_ref, *, TOPK):
    x = sel_ref[0]
    feat = x[:, 0:99]
    score = x[:, 99:100]
    pos_sel = x[:, 100:103]
    gate = jax.nn.sigmoid(score)
    x_ds = feat * gate
    h = jax.nn.relu(_dotd(x_ds, m1w_ref[...]) + m1b_ref[...])
    h = jax.nn.relu(_dotd(h, m2w_ref[...]) + m2b_ref[...])
    adj = _dotd(h, m3w_ref[...]) + m3b_ref[...]
    out_ref[0] = pos_sel + adj


def kernel(pos, W1, b1, W2, b2, W3, b3, pw, pb, m1w, m1b, m2w, m2b, m3w, m3b):
    B, N, _ = pos.shape
    K = 16
    R = 512
    TOPK = N // 2
    CH = 256

    posT = jnp.transpose(pos, (0, 2, 1))

    knn_idx = pl.pallas_call(
        functools.partial(_knn_extract_kernel, R=R, N=N, K=K),
        grid=(B, N // R),
        in_specs=[
            pl.BlockSpec((1, R, 3), lambda b, n: (b, n, 0)),
            pl.BlockSpec((1, 3, N), lambda b, n: (b, 0, 0)),
        ],
        out_specs=pl.BlockSpec((1, R, K), lambda b, n: (b, n, 0)),
        out_shape=jax.ShapeDtypeStruct((B, N, K), jnp.int32),
    )(pos, posT)

    idxT = jnp.transpose(knn_idx, (2, 0, 1)).reshape(K * B * N)
    pos128 = jnp.concatenate(
        [pos.reshape(B * N, 3), jnp.zeros((B * N, 125), jnp.float32)], axis=1)
    nbrows = _make_sc_gather_flat(K * B * N, 128)(idxT, pos128)
    nbrows = nbrows.reshape(K, B, N, 128)

    table = pl.pallas_call(
        functools.partial(_edgeconv_kernel, R=R, N=N, K=K),
        grid=(B, N // R),
        in_specs=[
            pl.BlockSpec((1, R, 3), lambda b, n: (b, n, 0)),
            pl.BlockSpec((K, 1, R, 128), lambda b, n: (0, b, n, 0)),
            pl.BlockSpec((9, 32), lambda b, n: (0, 0)),
            pl.BlockSpec((1, 32), lambda b, n: (0, 0)),
            pl.BlockSpec((35, 32), lambda b, n: (0, 0)),
            pl.BlockSpec((1, 32), lambda b, n: (0, 0)),
            pl.BlockSpec((67, 32), lambda b, n: (0, 0)),
            pl.BlockSpec((1, 32), lambda b, n: (0, 0)),
            pl.BlockSpec((99, 1), lambda b, n: (0, 0)),
            pl.BlockSpec((1, 1), lambda b, n: (0, 0)),
        ],
        out_specs=pl.BlockSpec((1, R, 128), lambda b, n: (b, n, 0)),
        out_shape=jax.ShapeDtypeStruct((B, N, 128), jnp.float32),
    )(pos, nbrows,
      W1, b1.reshape(1, 32), W2, b2.reshape(1, 32), W3, b3.reshape(1, 32),
      pw, pb.reshape(1, 1))

    score = table[:, :, 99]
    s_row = score.reshape(B, 1, N)
    s_col = score.reshape(B, N, 1)

    rank = pl.pallas_call(
        functools.partial(_rank_kernel, N=N, CH=CH),
        grid=(B,),
        in_specs=[
            pl.BlockSpec((1, 1, N), lambda b: (b, 0, 0)),
            pl.BlockSpec((1, N, 1), lambda b: (b, 0, 0)),
        ],
        out_specs=pl.BlockSpec((1, 1, N), lambda b: (b, 0, 0)),
        out_shape=jax.ShapeDtypeStruct((B, 1, N), jnp.int32),
    )(s_row, s_col)

    inv = _make_sc_scatter(B, N)(rank.reshape(B * N))
    sel = _make_sc_gather(B, N, TOPK, 128)(inv, table.reshape(B * N, 128))
    sel = sel.reshape(B, TOPK, 128)

    out = pl.pallas_call(
        functools.partial(_gate_adjust_kernel, TOPK=TOPK),
        grid=(B,),
        in_specs=[
            pl.BlockSpec((1, TOPK, 128), lambda b: (b, 0, 0)),
            pl.BlockSpec((99, 49), lambda b: (0, 0)),
            pl.BlockSpec((1, 49), lambda b: (0, 0)),
            pl.BlockSpec((49, 24), lambda b: (0, 0)),
            pl.BlockSpec((1, 24), lambda b: (0, 0)),
            pl.BlockSpec((24, 3), lambda b: (0, 0)),
            pl.BlockSpec((1, 3), lambda b: (0, 0)),
        ],
        out_specs=pl.BlockSpec((1, TOPK, 3), lambda b: (b, 0, 0)),
        out_shape=jax.ShapeDtypeStruct((B, TOPK, 3), jnp.float32),
    )(sel, m1w, m1b.reshape(1, 49), m2w, m2b.reshape(1, 24),
      m3w, m3b.reshape(1, 3))

    return out


# speedup vs baseline: 2.8532x; 1.0000x over previous
import functools

import jax
import jax.numpy as jnp
from jax import lax
from jax.experimental import pallas as pl
from jax.experimental.pallas import tpu as pltpu
from jax.experimental.pallas import tpu_sc as plsc

def _dotd(a, b):
    return jax.lax.dot_general(a, b, (((1,), (0,)), ((), ())),
                               precision=jax.lax.Precision.DEFAULT,
                               preferred_element_type=jnp.float32)


def _knn_extract_kernel(pos_blk_ref, posT_ref, idx_ref, *, R, N, K):
    q = pos_blk_ref[0]

    d = jnp.zeros((R, N), dtype=jnp.float32)
    for c in range(3):
        qc = q[:, c:c + 1]
        pc = posT_ref[0, c, :].reshape(1, N)
        diff = qc - pc
        d = d + diff * diff

    col = jax.lax.broadcasted_iota(jnp.int32, (R, N), 1)
    boff = pl.program_id(0) * N

    for k in range(K + 1):
        m = jnp.min(d, axis=1, keepdims=True)
        cand = jnp.where(d == m, col, N)
        idx = jnp.min(cand, axis=1, keepdims=True)
        d = jnp.where(col == idx, jnp.inf, d)
        if k == 0:
            continue
        idx_ref[0, :, k - 1:k] = idx + boff


def _edgeconv_kernel(pos_blk_ref, nb_ref,
                     W1_ref, b1_ref, W2_ref, b2_ref, W3_ref, b3_ref,
                     pw_ref, pb_ref, out_ref, *, R, N, K):
    q = pos_blk_ref[0]
    W1 = W1_ref[...]
    b1 = b1_ref[...]
    W2 = W2_ref[...]
    b2 = b2_ref[...]
    W3 = W3_ref[...]
    b3 = b3_ref[...]

    nb = nb_ref[:, 0, :, 0:3].reshape(K * R, 3)
    qt = jnp.broadcast_to(q[None], (K, R, 3)).reshape(K * R, 3)
    e = jnp.concatenate([qt, nb, nb - qt], axis=1)
    y1r = jax.nn.relu(_dotd(e, W1) + b1)
    y1 = jnp.concatenate([y1r, qt], axis=1)
    y2r = jax.nn.relu(_dotd(y1, W2) + b2)
    y2 = jnp.concatenate([y2r, y1], axis=1)
    l3 = _dotd(y2, W3) + b3
    y3 = jnp.concatenate([l3, y2], axis=1)
    fmax = jnp.max(y3.reshape(K, R, 99), axis=0)

    pw = pw_ref[...]
    norm = jnp.sqrt(jnp.sum(pw * pw))
    score = (_dotd(fmax, pw) + pb_ref[0, 0]) / norm
    pad = jnp.zeros((R, 25), dtype=jnp.float32)
    out_ref[0] = jnp.concatenate([fmax, score, q, pad], axis=1)


def _rank_kernel(srow_ref, scol_ref, rank_ref, *, N, CH):
    s_row = srow_ref[0]
    irow = jax.lax.broadcasted_iota(jnp.int32, (1, N), 1)
    rank = jnp.zeros((1, N), dtype=jnp.int32)
    for c in range(N // CH):
        s_col = scol_ref[0, c * CH:(c + 1) * CH, :]
        jcol = (jax.lax.broadcasted_iota(jnp.int32, (CH, 1), 0)
                + c * CH)
        gt = s_col > s_row
        eq = (s_col == s_row) & (jcol < irow)
        cnt = (gt | eq).astype(jnp.int32)
        rank = rank + jnp.sum(cnt, axis=0, keepdims=True)
    rank_ref[0] = rank


def _make_sc_scatter(B, N):
    mesh = plsc.VectorSubcoreMesh(core_axis_name="c", subcore_axis_name="s")
    per_w = (B * N) // 32
    n_ch = per_w // 128

    @functools.partial(
        pl.kernel, mesh=mesh,
        out_type=jax.ShapeDtypeStruct((B * N,), jnp.int32),
        scratch_types=[
            pltpu.VMEM((128,), jnp.int32),
            pltpu.VMEM((128,), jnp.int32),
            pltpu.SemaphoreType.DMA,
        ],
    )
    def sc_scatter(rank_hbm, inv_hbm, idx_v, val_v, sem):
        wid = lax.axis_index("s") * 2 + lax.axis_index("c")
        base = wid * per_w
        for ch in range(n_ch):
            off = base + ch * 128
            pltpu.sync_copy(rank_hbm.at[pl.ds(off, 128)], idx_v)
            b = (off // N) * N
            for j in range(8):
                lane = jax.lax.iota(jnp.int32, 16)
                val_v[pl.ds(j * 16, 16)] = lane + (off + j * 16)
                idx_v[pl.ds(j * 16, 16)] = idx_v[pl.ds(j * 16, 16)] + b
            pltpu.async_copy(val_v, inv_hbm.at[idx_v], sem).wait()

    return sc_scatter


def _make_sc_gather_flat(M, D):
    mesh = plsc.VectorSubcoreMesh(core_axis_name="c", subcore_axis_name="s")
    per_w = M // 32

    @functools.partial(
        pl.kernel, mesh=mesh,
        out_type=jax.ShapeDtypeStruct((M, D), jnp.float32),
        scratch_types=[
            pltpu.VMEM((128,), jnp.int32),
            pltpu.VMEM((128, D), jnp.float32),
            pltpu.SemaphoreType.DMA,
        ],
    )
    def sc_gather_flat(idx_hbm, table_hbm, out_hbm, idx_v, rows_v, sem):
        wid = lax.axis_index("s") * 2 + lax.axis_index("c")
        base = wid * per_w

        def body(ch, carry):
            o = base + ch * 128
            pltpu.sync_copy(idx_hbm.at[pl.ds(o, 128)], idx_v)
            pltpu.async_copy(table_hbm.at[idx_v], rows_v, sem).wait()
            pltpu.sync_copy(rows_v, out_hbm.at[pl.ds(o, 128)])
            return carry

        lax.fori_loop(0, per_w // 128, body, 0)

    return sc_gather_flat


def _make_sc_gather(B, N, TOPK, D):
    mesh = plsc.VectorSubcoreMesh(core_axis_name="c", subcore_axis_name="s")
    per_w = (B * TOPK) // 32
    n_ch = per_w // 128

    @functools.partial(
        pl.kernel, mesh=mesh,
        out_type=jax.ShapeDtypeStruct((B * TOPK, D), jnp.float32),
        scratch_types=[
            pltpu.VMEM((128,), jnp.int32),
            pltpu.VMEM((128, D), jnp.float32),
            pltpu.SemaphoreType.DMA,
        ],
    )
    def sc_gather(inv_hbm, table_hbm, out_hbm, idx_v, rows_v, sem):
        wid = lax.axis_index("s") * 2 + lax.axis_index("c")
        for ch in range(n_ch):
            o = wid * per_w + ch * 128
            b = o // TOPK
            r = o - b * TOPK
            pltpu.sync_copy(inv_hbm.at[pl.ds(b * N + r, 128)], idx_v)
            pltpu.async_copy(table_hbm.at[idx_v], rows_v, sem).wait()
            pltpu.sync_copy(rows_v, out_hbm.at[pl.ds(o, 128)])

    return sc_gather


def _gate_adjust_kernel(sel_ref, m1w_ref, m1b_ref, m2w_ref, m2b_ref,
                        m3w_ref, m3b_ref, out_ref, *, TOPK):
    x = sel_ref[0]
    feat = x[:, 0:99]
    score = x[:, 99:100]
    pos_sel = x[:, 100:103]
    gate = jax.nn.sigmoid(score)
    x_ds = feat * gate
    h = jax.nn.relu(_dotd(x_ds, m1w_ref[...]) + m1b_ref[...])
    h = jax.nn.relu(_dotd(h, m2w_ref[...]) + m2b_ref[...])
    adj = _dotd(h, m3w_ref[...]) + m3b_ref[...]
    out_ref[0] = pos_sel + adj


def kernel(pos, W1, b1, W2, b2, W3, b3, pw, pb, m1w, m1b, m2w, m2b, m3w, m3b):
    B, N, _ = pos.shape
    K = 16
    R = 512
    TOPK = N // 2
    CH = 256

    posT = jnp.transpose(pos, (0, 2, 1))

    knn_idx = pl.pallas_call(
        functools.partial(_knn_extract_kernel, R=R, N=N, K=K),
        grid=(B, N // R),
        in_specs=[
            pl.BlockSpec((1, R, 3), lambda b, n: (b, n, 0)),
            pl.BlockSpec((1, 3, N), lambda b, n: (b, 0, 0)),
        ],
        out_specs=pl.BlockSpec((1, R, K), lambda b, n: (b, n, 0)),
        out_shape=jax.ShapeDtypeStruct((B, N, K), jnp.int32),
    )(pos, posT)

    idxT = jnp.transpose(knn_idx, (2, 0, 1)).reshape(K * B * N)
    pos128 = jnp.concatenate(
        [pos.reshape(B * N, 3), jnp.zeros((B * N, 125), jnp.float32)], axis=1)
    nbrows = _make_sc_gather_flat(K * B * N, 128)(idxT, pos128)
    nbrows = nbrows.reshape(K, B, N, 128)

    table = pl.pallas_call(
        functools.partial(_edgeconv_kernel, R=R, N=N, K=K),
        grid=(B, N // R),
        in_specs=[
            pl.BlockSpec((1, R, 3), lambda b, n: (b, n, 0)),
            pl.BlockSpec((K, 1, R, 128), lambda b, n: (0, b, n, 0)),
            pl.BlockSpec((9, 32), lambda b, n: (0, 0)),
            pl.BlockSpec((1, 32), lambda b, n: (0, 0)),
            pl.BlockSpec((35, 32), lambda b, n: (0, 0)),
            pl.BlockSpec((1, 32), lambda b, n: (0, 0)),
            pl.BlockSpec((67, 32), lambda b, n: (0, 0)),
            pl.BlockSpec((1, 32), lambda b, n: (0, 0)),
            pl.BlockSpec((99, 1), lambda b, n: (0, 0)),
            pl.BlockSpec((1, 1), lambda b, n: (0, 0)),
        ],
        out_specs=pl.BlockSpec((1, R, 128), lambda b, n: (b, n, 0)),
        out_shape=jax.ShapeDtypeStruct((B, N, 128), jnp.float32),
    )(pos, nbrows,
      W1, b1.reshape(1, 32), W2, b2.reshape(1, 32), W3, b3.reshape(1, 32),
      pw, pb.reshape(1, 1))

    score = table[:, :, 99]
    s_row = score.reshape(B, 1, N)
    s_col = score.reshape(B, N, 1)

    rank = pl.pallas_call(
        functools.partial(_rank_kernel, N=N, CH=CH),
        grid=(B,),
        in_specs=[
            pl.BlockSpec((1, 1, N), lambda b: (b, 0, 0)),
            pl.BlockSpec((1, N, 1), lambda b: (b, 0, 0)),
        ],
        out_specs=pl.BlockSpec((1, 1, N), lambda b: (b, 0, 0)),
        out_shape=jax.ShapeDtypeStruct((B, 1, N), jnp.int32),
    )(s_row, s_col)

    inv = _make_sc_scatter(B, N)(rank.reshape(B * N))
    sel = _make_sc_gather(B, N, TOPK, 128)(inv, table.reshape(B * N, 128))
    sel = sel.reshape(B, TOPK, 128)

    out = pl.pallas_call(
        functools.partial(_gate_adjust_kernel, TOPK=TOPK),
        grid=(B,),
        in_specs=[
            pl.BlockSpec((1, TOPK, 128), lambda b: (b, 0, 0)),
            pl.BlockSpec((99, 49), lambda b: (0, 0)),
            pl.BlockSpec((1, 49), lambda b: (0, 0)),
            pl.BlockSpec((49, 24), lambda b: (0, 0)),
            pl.BlockSpec((1, 24), lambda b: (0, 0)),
            pl.BlockSpec((24, 3), lambda b: (0, 0)),
            pl.BlockSpec((1, 3), lambda b: (0, 0)),
        ],
        out_specs=pl.BlockSpec((1, TOPK, 3), lambda b: (b, 0, 0)),
        out_shape=jax.ShapeDtypeStruct((B, TOPK, 3), jnp.float32),
    )(sel, m1w, m1b.reshape(1, 49), m2w, m2b.reshape(1, 24),
      m3w, m3b.reshape(1, 3))

    return out
